# Initial kernel scaffold; baseline (speedup 1.0000x reference)
import jax
import jax.numpy as jnp
from jax.experimental import pallas as pl


def kernel(x, edge_index, emb1, emb3, learnable_x, cond_Wi, cond_bi, cond_Wo, cond_bo, g_W1, g_b1, g_W2, g_b2, c_W1, c_b1, c_W2, c_b2):
    raise NotImplementedError("write your pallas kernel here")


# speedup vs baseline: 14.1655x; 14.1655x over previous
import functools

import jax
import jax.numpy as jnp
from jax import lax
from jax.experimental import pallas as pl
from jax.experimental.pallas import tpu as pltpu
from jax.experimental.pallas import tpu_sc as plsc

_N = 10000
_E = 320000
_D = 128

_NSC = 2
_NSUB = 16
_NW = _NSC * _NSUB
_EPT = _E // _NW
_CH = 128
_NFULL = _EPT // _CH
_TAIL = _EPT - _NFULL * _CH
_RPT = 624

_mesh = plsc.VectorSubcoreMesh(core_axis_name="c", subcore_axis_name="s")


@functools.partial(
    pl.kernel,
    out_type=jax.ShapeDtypeStruct((_NSC, _N, 16), jnp.float32),
    mesh=_mesh,
    scratch_types=[
        pltpu.VMEM((_CH,), jnp.int32),
        pltpu.VMEM((_TAIL,), jnp.int32),
        pltpu.VMEM((_CH, 16), jnp.float32),
        pltpu.VMEM((48, 16), jnp.float32),
        pltpu.VMEM_SHARED((_N, 16), jnp.float32),
    ],
)
def _deg_kernel(dst_hbm, out_hbm, idx_v, idxt_v, ones_v, zer_v, acc_sh):
    cid = lax.axis_index("c")
    sid = lax.axis_index("s")
    wid = sid * _NSC + cid

    @pl.loop(0, 48)
    def _(i):
        zer_v[i, :] = jnp.zeros((16,), jnp.float32)

    @pl.loop(0, _CH)
    def _(i):
        ones_v[i, :] = jnp.ones((16,), jnp.float32)

    r0 = sid * _RPT

    @pl.loop(0, 13)
    def _(j):
        pltpu.sync_copy(zer_v, acc_sh.at[pl.ds(r0 + j * 48, 48)])

    @pl.when(sid == _NSUB - 1)
    def _():
        pltpu.sync_copy(zer_v.at[pl.ds(0, 16)], acc_sh.at[pl.ds(_N - 16, 16)])

    plsc.subcore_barrier()

    base = wid * _EPT

    @pl.loop(0, _NFULL)
    def _(i):
        pltpu.sync_copy(dst_hbm.at[pl.ds(base + i * _CH, _CH)], idx_v)
        pltpu.sync_copy(ones_v, acc_sh.at[idx_v], add=True)

    pltpu.sync_copy(dst_hbm.at[pl.ds(base + _NFULL * _CH, _TAIL)], idxt_v)
    pltpu.sync_copy(ones_v.at[pl.ds(0, _TAIL)], acc_sh.at[idxt_v], add=True)

    plsc.subcore_barrier()

    @pl.loop(0, 13)
    def _(j):
        rr = r0 + j * 48
        pltpu.sync_copy(acc_sh.at[pl.ds(rr, 48)],
                        out_hbm.at[cid, pl.ds(rr, 48), :])

    @pl.when(sid == _NSUB - 1)
    def _():
        pltpu.sync_copy(acc_sh.at[pl.ds(_N - 16, 16)],
                        out_hbm.at[cid, pl.ds(_N - 16, 16), :])


@functools.partial(
    pl.kernel,
    out_type=jax.ShapeDtypeStruct((_NSC, _N, _D), jnp.float32),
    mesh=_mesh,
    scratch_types=[
        pltpu.VMEM((_CH,), jnp.int32),
        pltpu.VMEM((_CH,), jnp.int32),
        pltpu.VMEM((_TAIL,), jnp.int32),
        pltpu.VMEM((_TAIL,), jnp.int32),
        pltpu.VMEM((_CH, _D), jnp.float32),
        pltpu.VMEM((48, _D), jnp.float32),
        pltpu.VMEM_SHARED((_N, _D), jnp.float32),
        pltpu.SemaphoreType.DMA,
    ],
)
def _segsum_kernel(hs_hbm, src_hbm, dst_hbm, out_hbm,
                   si_v, di_v, sit_v, dit_v, rows_v, zer_v, acc_sh, sem):
    cid = lax.axis_index("c")
    sid = lax.axis_index("s")
    wid = sid * _NSC + cid

    @pl.loop(0, 48)
    def _(i):
        @pl.loop(0, _D // 16)
        def _(j):
            zer_v[i, pl.ds(j * 16, 16)] = jnp.zeros((16,), jnp.float32)

    r0 = sid * _RPT

    @pl.loop(0, 13)
    def _(j):
        pltpu.sync_copy(zer_v, acc_sh.at[pl.ds(r0 + j * 48, 48)])

    @pl.when(sid == _NSUB - 1)
    def _():
        pltpu.sync_copy(zer_v.at[pl.ds(0, 16)], acc_sh.at[pl.ds(_N - 16, 16)])

    plsc.subcore_barrier()

    base = wid * _EPT

    @pl.loop(0, _NFULL)
    def _(i):
        pltpu.sync_copy(src_hbm.at[pl.ds(base + i * _CH, _CH)], si_v)
        pltpu.sync_copy(dst_hbm.at[pl.ds(base + i * _CH, _CH)], di_v)
        pltpu.async_copy(hs_hbm.at[si_v], rows_v, sem).wait()
        pltpu.sync_copy(rows_v, acc_sh.at[di_v], add=True)

    pltpu.sync_copy(src_hbm.at[pl.ds(base + _NFULL * _CH, _TAIL)], sit_v)
    pltpu.sync_copy(dst_hbm.at[pl.ds(base + _NFULL * _CH, _TAIL)], dit_v)
    pltpu.async_copy(hs_hbm.at[sit_v], rows_v.at[pl.ds(0, _TAIL)], sem).wait()
    pltpu.sync_copy(rows_v.at[pl.ds(0, _TAIL)], acc_sh.at[dit_v], add=True)

    plsc.subcore_barrier()

    @pl.loop(0, 13)
    def _(j):
        rr = r0 + j * 48
        pltpu.sync_copy(acc_sh.at[pl.ds(rr, 48)],
                        out_hbm.at[cid, pl.ds(rr, 48), :])

    @pl.when(sid == _NSUB - 1)
    def _():
        pltpu.sync_copy(acc_sh.at[pl.ds(_N - 16, 16)],
                        out_hbm.at[cid, pl.ds(_N - 16, 16), :])


_BLK = 1000


def _tc1_body(x_ref, lx_ref, w_ref, b_ref, xo_ref, h_ref):
    xo = x_ref[...] * lx_ref[...]
    xo_ref[...] = xo
    h_ref[...] = (jnp.dot(xo, w_ref[...], preferred_element_type=jnp.float32)
                  + b_ref[...])


def _tc2_body(d0_ref, d1_ref, h_ref, hs_ref):
    dinv = lax.rsqrt(d0_ref[...] + d1_ref[...] + 1.0)
    hs_ref[...] = h_ref[...] * dinv


def _tc3_body(d0_ref, d1_ref, s0_ref, s1_ref, hs_ref, w_ref, b_ref, out_ref):
    dinv = lax.rsqrt(d0_ref[...] + d1_ref[...] + 1.0)
    t = (s0_ref[...] + s1_ref[...] + hs_ref[...]) * dinv
    h1 = jnp.maximum(t, 0.0)
    out_ref[...] = (jnp.dot(h1, w_ref[...], preferred_element_type=jnp.float32)
                    + b_ref[...]) * dinv


def _tc4_body(d0_ref, d1_ref, s0_ref, s1_ref, hs_ref,
              w1_ref, b1_ref, w2_ref, b2_ref, out_ref):
    dinv = lax.rsqrt(d0_ref[...] + d1_ref[...] + 1.0)
    embed = (s0_ref[...] + s1_ref[...] + hs_ref[...]) * dinv
    hidden = jnp.maximum(
        jnp.dot(embed, w1_ref[...], preferred_element_type=jnp.float32)
        + b1_ref[...], 0.0)
    out_ref[...] = (jnp.dot(hidden, w2_ref[...],
                            preferred_element_type=jnp.float32) + b2_ref[...])


def _row_spec():
    return pl.BlockSpec((_BLK, _D), lambda i: (i, 0))


def _deg_spec():
    return pl.BlockSpec((_BLK, 1), lambda i: (i, 0))


def _full_spec(shape):
    return pl.BlockSpec(shape, lambda i: tuple(0 for _ in shape))


def kernel(x, edge_index, emb1, emb3, learnable_x, cond_Wi, cond_bi, cond_Wo,
           cond_bo, g_W1, g_b1, g_W2, g_b2, c_W1, c_b1, c_W2, c_b2):
    src = edge_index[0]
    dst = edge_index[1]
    n, d = x.shape
    grid = (n // _BLK,)

    degp = _deg_kernel(dst)
    d0 = degp[0, :, 0].reshape(n, 1)
    d1 = degp[1, :, 0].reshape(n, 1)

    x_out, h1_pre = pl.pallas_call(
        _tc1_body,
        grid=grid,
        in_specs=[_row_spec(), _row_spec(),
                  _full_spec((_D, _D)), _full_spec((1, _D))],
        out_specs=[_row_spec(), _row_spec()],
        out_shape=[jax.ShapeDtypeStruct((n, d), jnp.float32)] * 2,
    )(x, learnable_x, g_W1, g_b1.reshape(1, d))

    hs1 = pl.pallas_call(
        _tc2_body,
        grid=grid,
        in_specs=[_deg_spec(), _deg_spec(), _row_spec()],
        out_specs=_row_spec(),
        out_shape=jax.ShapeDtypeStruct((n, d), jnp.float32),
    )(d0, d1, h1_pre)

    s1 = _segsum_kernel(hs1, src, dst)

    hs2 = pl.pallas_call(
        _tc3_body,
        grid=grid,
        in_specs=[_deg_spec(), _deg_spec(), _row_spec(), _row_spec(),
                  _row_spec(), _full_spec((_D, _D)), _full_spec((1, _D))],
        out_specs=_row_spec(),
        out_shape=jax.ShapeDtypeStruct((n, d), jnp.float32),
    )(d0, d1, s1[0], s1[1], hs1, g_W2, g_b2.reshape(1, d))

    s2 = _segsum_kernel(hs2, src, dst)

    nh = c_W1.shape[1]
    nc = c_W2.shape[1]
    logits = pl.pallas_call(
        _tc4_body,
        grid=grid,
        in_specs=[_deg_spec(), _deg_spec(), _row_spec(), _row_spec(),
                  _row_spec(), _full_spec((_D, nh)), _full_spec((1, nh)),
                  _full_spec((nh, nc)), _full_spec((1, nc))],
        out_specs=pl.BlockSpec((_BLK, nc), lambda i: (i, 0)),
        out_shape=jax.ShapeDtypeStruct((n, nc), jnp.float32),
    )(d0, d1, s2[0], s2[1], hs2, c_W1, c_b1.reshape(1, nh),
      c_W2, c_b2.reshape(1, nc))

    return (x_out, logits)


# speedup vs baseline: 19.9429x; 1.4079x over previous
import functools

import jax
import jax.numpy as jnp
from jax import lax
from jax.experimental import pallas as pl
from jax.experimental.pallas import tpu as pltpu
from jax.experimental.pallas import tpu_sc as plsc

_N = 10000
_E = 320000
_D = 128

_NSC = 2
_NSUB = 16
_NW = _NSC * _NSUB
_EPT = _E // _NW
_CH = 128
_NFULL = _EPT // _CH
_TAIL = _EPT - _NFULL * _CH
_RPT = 624

_mesh = plsc.VectorSubcoreMesh(core_axis_name="c", subcore_axis_name="s")


@functools.partial(
    pl.kernel,
    out_type=jax.ShapeDtypeStruct((_NSC, _N, 16), jnp.float32),
    mesh=_mesh,
    scratch_types=[
        pltpu.VMEM((_CH,), jnp.int32),
        pltpu.VMEM((_TAIL,), jnp.int32),
        pltpu.VMEM((_CH, 16), jnp.float32),
        pltpu.VMEM((48, 16), jnp.float32),
        pltpu.VMEM_SHARED((_N, 16), jnp.float32),
    ],
)
def _deg_kernel(dst_hbm, out_hbm, idx_v, idxt_v, ones_v, zer_v, acc_sh):
    cid = lax.axis_index("c")
    sid = lax.axis_index("s")
    wid = sid * _NSC + cid

    @pl.loop(0, 48)
    def _(i):
        zer_v[i, :] = jnp.zeros((16,), jnp.float32)

    @pl.loop(0, _CH)
    def _(i):
        ones_v[i, :] = jnp.ones((16,), jnp.float32)

    r0 = sid * _RPT

    @pl.loop(0, 13)
    def _(j):
        pltpu.sync_copy(zer_v, acc_sh.at[pl.ds(r0 + j * 48, 48)])

    @pl.when(sid == _NSUB - 1)
    def _():
        pltpu.sync_copy(zer_v.at[pl.ds(0, 16)], acc_sh.at[pl.ds(_N - 16, 16)])

    plsc.subcore_barrier()

    base = wid * _EPT

    @pl.loop(0, _NFULL)
    def _(i):
        pltpu.sync_copy(dst_hbm.at[pl.ds(base + i * _CH, _CH)], idx_v)
        pltpu.sync_copy(ones_v, acc_sh.at[idx_v], add=True)

    pltpu.sync_copy(dst_hbm.at[pl.ds(base + _NFULL * _CH, _TAIL)], idxt_v)
    pltpu.sync_copy(ones_v.at[pl.ds(0, _TAIL)], acc_sh.at[idxt_v], add=True)

    plsc.subcore_barrier()

    @pl.loop(0, 13)
    def _(j):
        rr = r0 + j * 48
        pltpu.sync_copy(acc_sh.at[pl.ds(rr, 48)],
                        out_hbm.at[cid, pl.ds(rr, 48), :])

    @pl.when(sid == _NSUB - 1)
    def _():
        pltpu.sync_copy(acc_sh.at[pl.ds(_N - 16, 16)],
                        out_hbm.at[cid, pl.ds(_N - 16, 16), :])


@functools.partial(
    pl.kernel,
    out_type=jax.ShapeDtypeStruct((_NSC, _N, _D), jnp.float32),
    mesh=_mesh,
    scratch_types=[
        pltpu.VMEM((_CH,), jnp.int32),
        pltpu.VMEM((_CH,), jnp.int32),
        pltpu.VMEM((_CH,), jnp.int32),
        pltpu.VMEM((_CH,), jnp.int32),
        pltpu.VMEM((_TAIL,), jnp.int32),
        pltpu.VMEM((_TAIL,), jnp.int32),
        pltpu.VMEM((_CH, _D), jnp.float32),
        pltpu.VMEM((_CH, _D), jnp.float32),
        pltpu.VMEM((48, _D), jnp.float32),
        pltpu.VMEM_SHARED((_N, _D), jnp.float32),
        pltpu.SemaphoreType.DMA,
        pltpu.SemaphoreType.DMA,
    ],
)
def _segsum_kernel(hs_hbm, src_hbm, dst_hbm, out_hbm,
                   siA, diA, siB, diB, sit_v, dit_v, rA, rB,
                   zer_v, acc_sh, semA, semB):
    cid = lax.axis_index("c")
    sid = lax.axis_index("s")
    wid = sid * _NSC + cid

    @pl.loop(0, 48)
    def _(i):
        @pl.loop(0, _D // 16)
        def _(j):
            zer_v[i, pl.ds(j * 16, 16)] = jnp.zeros((16,), jnp.float32)

    r0 = sid * _RPT

    @pl.loop(0, 13)
    def _(j):
        pltpu.sync_copy(zer_v, acc_sh.at[pl.ds(r0 + j * 48, 48)])

    @pl.when(sid == _NSUB - 1)
    def _():
        pltpu.sync_copy(zer_v.at[pl.ds(0, 16)], acc_sh.at[pl.ds(_N - 16, 16)])

    plsc.subcore_barrier()

    base = wid * _EPT

    def fetch_idx(c, si, di):
        pltpu.sync_copy(src_hbm.at[pl.ds(base + c * _CH, _CH)], si)
        pltpu.sync_copy(dst_hbm.at[pl.ds(base + c * _CH, _CH)], di)

    def gather(si, rows, sem):
        return pltpu.make_async_copy(hs_hbm.at[si], rows, sem)

    def scatter(rows, di):
        pltpu.sync_copy(rows, acc_sh.at[di], add=True)

    fetch_idx(0, siA, diA)
    gather(siA, rA, semA).start()

    @pl.loop(0, _NFULL, step=2)
    def _(i):
        fetch_idx(i + 1, siB, diB)
        gather(siA, rA, semA).wait()
        gather(siB, rB, semB).start()
        scatter(rA, diA)

        @pl.when(i + 2 < _NFULL)
        def _():
            fetch_idx(i + 2, siA, diA)
        gather(siB, rB, semB).wait()

        @pl.when(i + 2 < _NFULL)
        def _():
            gather(siA, rA, semA).start()
        scatter(rB, diB)

    pltpu.sync_copy(src_hbm.at[pl.ds(base + _NFULL * _CH, _TAIL)], sit_v)
    pltpu.sync_copy(dst_hbm.at[pl.ds(base + _NFULL * _CH, _TAIL)], dit_v)
    pltpu.async_copy(hs_hbm.at[sit_v], rA.at[pl.ds(0, _TAIL)], semA).wait()
    pltpu.sync_copy(rA.at[pl.ds(0, _TAIL)], acc_sh.at[dit_v], add=True)

    plsc.subcore_barrier()

    @pl.loop(0, 13)
    def _(j):
        rr = r0 + j * 48
        pltpu.sync_copy(acc_sh.at[pl.ds(rr, 48)],
                        out_hbm.at[cid, pl.ds(rr, 48), :])

    @pl.when(sid == _NSUB - 1)
    def _():
        pltpu.sync_copy(acc_sh.at[pl.ds(_N - 16, 16)],
                        out_hbm.at[cid, pl.ds(_N - 16, 16), :])


_BLK = 1000


def _tc1_body(x_ref, lx_ref, w_ref, b_ref, xo_ref, h_ref):
    xo = x_ref[...] * lx_ref[...]
    xo_ref[...] = xo
    h_ref[...] = (jnp.dot(xo, w_ref[...], preferred_element_type=jnp.float32)
                  + b_ref[...])


def _tc2_body(d0_ref, d1_ref, h_ref, hs_ref):
    dinv = lax.rsqrt(d0_ref[...] + d1_ref[...] + 1.0)
    hs_ref[...] = h_ref[...] * dinv


def _tc3_body(d0_ref, d1_ref, s0_ref, s1_ref, hs_ref, w_ref, b_ref, out_ref):
    dinv = lax.rsqrt(d0_ref[...] + d1_ref[...] + 1.0)
    t = (s0_ref[...] + s1_ref[...] + hs_ref[...]) * dinv
    h1 = jnp.maximum(t, 0.0)
    out_ref[...] = (jnp.dot(h1, w_ref[...], preferred_element_type=jnp.float32)
                    + b_ref[...]) * dinv


def _tc4_body(d0_ref, d1_ref, s0_ref, s1_ref, hs_ref,
              w1_ref, b1_ref, w2_ref, b2_ref, out_ref):
    dinv = lax.rsqrt(d0_ref[...] + d1_ref[...] + 1.0)
    embed = (s0_ref[...] + s1_ref[...] + hs_ref[...]) * dinv
    hidden = jnp.maximum(
        jnp.dot(embed, w1_ref[...], preferred_element_type=jnp.float32)
        + b1_ref[...], 0.0)
    out_ref[...] = (jnp.dot(hidden, w2_ref[...],
                            preferred_element_type=jnp.float32) + b2_ref[...])


def _row_spec():
    return pl.BlockSpec((_BLK, _D), lambda i: (i, 0))


def _deg_spec():
    return pl.BlockSpec((_BLK, 1), lambda i: (i, 0))


def _full_spec(shape):
    return pl.BlockSpec(shape, lambda i: tuple(0 for _ in shape))


def kernel(x, edge_index, emb1, emb3, learnable_x, cond_Wi, cond_bi, cond_Wo,
           cond_bo, g_W1, g_b1, g_W2, g_b2, c_W1, c_b1, c_W2, c_b2):
    src = edge_index[0]
    dst = edge_index[1]
    n, d = x.shape
    grid = (n // _BLK,)

    degp = _deg_kernel(dst)
    d0 = degp[0, :, 0].reshape(n, 1)
    d1 = degp[1, :, 0].reshape(n, 1)

    x_out, h1_pre = pl.pallas_call(
        _tc1_body,
        grid=grid,
        in_specs=[_row_spec(), _row_spec(),
                  _full_spec((_D, _D)), _full_spec((1, _D))],
        out_specs=[_row_spec(), _row_spec()],
        out_shape=[jax.ShapeDtypeStruct((n, d), jnp.float32)] * 2,
    )(x, learnable_x, g_W1, g_b1.reshape(1, d))

    hs1 = pl.pallas_call(
        _tc2_body,
        grid=grid,
        in_specs=[_deg_spec(), _deg_spec(), _row_spec()],
        out_specs=_row_spec(),
        out_shape=jax.ShapeDtypeStruct((n, d), jnp.float32),
    )(d0, d1, h1_pre)

    s1 = _segsum_kernel(hs1, src, dst)

    hs2 = pl.pallas_call(
        _tc3_body,
        grid=grid,
        in_specs=[_deg_spec(), _deg_spec(), _row_spec(), _row_spec(),
                  _row_spec(), _full_spec((_D, _D)), _full_spec((1, _D))],
        out_specs=_row_spec(),
        out_shape=jax.ShapeDtypeStruct((n, d), jnp.float32),
    )(d0, d1, s1[0], s1[1], hs1, g_W2, g_b2.reshape(1, d))

    s2 = _segsum_kernel(hs2, src, dst)

    nh = c_W1.shape[1]
    nc = c_W2.shape[1]
    logits = pl.pallas_call(
        _tc4_body,
        grid=grid,
        in_specs=[_deg_spec(), _deg_spec(), _row_spec(), _row_spec(),
                  _row_spec(), _full_spec((_D, nh)), _full_spec((1, nh)),
                  _full_spec((nh, nc)), _full_spec((1, nc))],
        out_specs=pl.BlockSpec((_BLK, nc), lambda i: (i, 0)),
        out_shape=jax.ShapeDtypeStruct((n, nc), jnp.float32),
    )(d0, d1, s2[0], s2[1], hs2, c_W1, c_b1.reshape(1, nh),
      c_W2, c_b2.reshape(1, nc))

    return (x_out, logits)


# speedup vs baseline: 23.7516x; 1.1910x over previous
import functools

import jax
import jax.numpy as jnp
from jax import lax
from jax.experimental import pallas as pl
from jax.experimental.pallas import tpu as pltpu
from jax.experimental.pallas import tpu_sc as plsc

_N = 10000
_E = 320000
_D = 128

_NSC = 2
_NSUB = 16
_NW = _NSC * _NSUB
_CH = 128
_NCHK = _E // _CH
_CPT = _NCHK // _NW
_LEFT = _NCHK - _CPT * _NW
_RPT = 624

_mesh = plsc.VectorSubcoreMesh(core_axis_name="c", subcore_axis_name="s")


@functools.partial(
    pl.kernel,
    out_type=jax.ShapeDtypeStruct((_NSC, _N, 16), jnp.float32),
    mesh=_mesh,
    scratch_types=[
        pltpu.VMEM((2, _CH), jnp.int32),
        pltpu.VMEM((2, _CH), jnp.int32),
        pltpu.VMEM((_CH, 16), jnp.float32),
        pltpu.VMEM((48, 16), jnp.float32),
        pltpu.VMEM_SHARED((_N, 16), jnp.float32),
        pltpu.SemaphoreType.DMA,
        pltpu.SemaphoreType.DMA,
    ],
)
def _deg_kernel(ei_hbm, out_hbm, eiA, eiB, ones_v, zer_v, acc_sh,
                semIA, semIB):
    cid = lax.axis_index("c")
    sid = lax.axis_index("s")
    wid = sid * _NSC + cid

    @pl.loop(0, 48)
    def _(i):
        zer_v[i, :] = jnp.zeros((16,), jnp.float32)

    @pl.loop(0, _CH)
    def _(i):
        ones_v[i, :] = jnp.ones((16,), jnp.float32)

    r0 = sid * _RPT

    @pl.loop(0, 13)
    def _(j):
        pltpu.sync_copy(zer_v, acc_sh.at[pl.ds(r0 + j * 48, 48)])

    @pl.when(sid == _NSUB - 1)
    def _():
        pltpu.sync_copy(zer_v.at[pl.ds(0, 16)], acc_sh.at[pl.ds(_N - 16, 16)])

    plsc.subcore_barrier()

    base = wid * _CPT

    def fetch_idx(c, ei, sem):
        return pltpu.make_async_copy(
            ei_hbm.at[:, pl.ds(c * _CH, _CH)], ei, sem)

    def scatter_ones(ei):
        pltpu.sync_copy(ones_v, acc_sh.at[ei.at[1]], add=True)

    fetch_idx(base, eiA, semIA).start()
    fetch_idx(base + 1, eiB, semIB).start()

    @pl.loop(0, _CPT, step=2)
    def _(t):
        fetch_idx(base + t, eiA, semIA).wait()
        scatter_ones(eiA)

        @pl.when(t + 2 < _CPT)
        def _():
            fetch_idx(base + t + 2, eiA, semIA).start()

        fetch_idx(base + t + 1, eiB, semIB).wait()
        scatter_ones(eiB)

        @pl.when(t + 3 < _CPT)
        def _():
            fetch_idx(base + t + 3, eiB, semIB).start()

    @pl.when(wid < _LEFT)
    def _():
        c = _NW * _CPT + wid
        fetch_idx(c, eiA, semIA).start()
        fetch_idx(c, eiA, semIA).wait()
        scatter_ones(eiA)

    plsc.subcore_barrier()

    @pl.loop(0, 13)
    def _(j):
        rr = r0 + j * 48
        pltpu.sync_copy(acc_sh.at[pl.ds(rr, 48)],
                        out_hbm.at[cid, pl.ds(rr, 48), :])

    @pl.when(sid == _NSUB - 1)
    def _():
        pltpu.sync_copy(acc_sh.at[pl.ds(_N - 16, 16)],
                        out_hbm.at[cid, pl.ds(_N - 16, 16), :])


@functools.partial(
    pl.kernel,
    out_type=jax.ShapeDtypeStruct((_NSC, _N, _D), jnp.float32),
    mesh=_mesh,
    scratch_types=[
        pltpu.VMEM((2, _CH), jnp.int32),
        pltpu.VMEM((2, _CH), jnp.int32),
        pltpu.VMEM((_CH, _D), jnp.float32),
        pltpu.VMEM((_CH, _D), jnp.float32),
        pltpu.VMEM((48, _D), jnp.float32),
        pltpu.VMEM_SHARED((_N, _D), jnp.float32),
        pltpu.SemaphoreType.DMA,
        pltpu.SemaphoreType.DMA,
        pltpu.SemaphoreType.DMA,
        pltpu.SemaphoreType.DMA,
    ],
)
def _segsum_kernel(hs_hbm, ei_hbm, out_hbm,
                   eiA, eiB, rA, rB, zer_v, acc_sh,
                   semA, semB, semIA, semIB):
    cid = lax.axis_index("c")
    sid = lax.axis_index("s")
    wid = sid * _NSC + cid

    @pl.loop(0, 48)
    def _(i):
        @pl.loop(0, _D // 16)
        def _(j):
            zer_v[i, pl.ds(j * 16, 16)] = jnp.zeros((16,), jnp.float32)

    r0 = sid * _RPT

    @pl.loop(0, 13)
    def _(j):
        pltpu.sync_copy(zer_v, acc_sh.at[pl.ds(r0 + j * 48, 48)])

    @pl.when(sid == _NSUB - 1)
    def _():
        pltpu.sync_copy(zer_v.at[pl.ds(0, 16)], acc_sh.at[pl.ds(_N - 16, 16)])

    plsc.subcore_barrier()

    base = wid * _CPT

    def fetch_idx(c, ei, sem):
        return pltpu.make_async_copy(
            ei_hbm.at[:, pl.ds(c * _CH, _CH)], ei, sem)

    def gather(ei, rows, sem):
        return pltpu.make_async_copy(hs_hbm.at[ei.at[0]], rows, sem)

    def scatter(rows, ei):
        pltpu.sync_copy(rows, acc_sh.at[ei.at[1]], add=True)

    fetch_idx(base, eiA, semIA).start()
    fetch_idx(base, eiA, semIA).wait()
    gather(eiA, rA, semA).start()
    fetch_idx(base + 1, eiB, semIB).start()

    @pl.loop(0, _CPT, step=2)
    def _(t):
        gather(eiA, rA, semA).wait()
        fetch_idx(base + t + 1, eiB, semIB).wait()
        gather(eiB, rB, semB).start()
        scatter(rA, eiA)

        @pl.when(t + 2 < _CPT)
        def _():
            fetch_idx(base + t + 2, eiA, semIA).start()

        gather(eiB, rB, semB).wait()

        @pl.when(t + 2 < _CPT)
        def _():
            fetch_idx(base + t + 2, eiA, semIA).wait()
            gather(eiA, rA, semA).start()
        scatter(rB, eiB)

        @pl.when(t + 3 < _CPT)
        def _():
            fetch_idx(base + t + 3, eiB, semIB).start()

    @pl.when(wid < _LEFT)
    def _():
        c = _NW * _CPT + wid
        fetch_idx(c, eiA, semIA).start()
        fetch_idx(c, eiA, semIA).wait()
        pltpu.async_copy(hs_hbm.at[eiA.at[0]], rA, semA).wait()
        scatter(rA, eiA)

    plsc.subcore_barrier()

    @pl.loop(0, 13)
    def _(j):
        rr = r0 + j * 48
        pltpu.sync_copy(acc_sh.at[pl.ds(rr, 48)],
                        out_hbm.at[cid, pl.ds(rr, 48), :])

    @pl.when(sid == _NSUB - 1)
    def _():
        pltpu.sync_copy(acc_sh.at[pl.ds(_N - 16, 16)],
                        out_hbm.at[cid, pl.ds(_N - 16, 16), :])


_BLK = 1000


def _tc1_body(x_ref, lx_ref, w_ref, b_ref, xo_ref, h_ref):
    xo = x_ref[...] * lx_ref[...]
    xo_ref[...] = xo
    h_ref[...] = (jnp.dot(xo, w_ref[...], preferred_element_type=jnp.float32)
                  + b_ref[...])


def _tc2_body(d0_ref, d1_ref, h_ref, hs_ref):
    dinv = lax.rsqrt(d0_ref[...] + d1_ref[...] + 1.0)
    hs_ref[...] = h_ref[...] * dinv


def _tc3_body(d0_ref, d1_ref, s0_ref, s1_ref, hs_ref, w_ref, b_ref, out_ref):
    dinv = lax.rsqrt(d0_ref[...] + d1_ref[...] + 1.0)
    t = (s0_ref[...] + s1_ref[...] + hs_ref[...]) * dinv
    h1 = jnp.maximum(t, 0.0)
    out_ref[...] = (jnp.dot(h1, w_ref[...], preferred_element_type=jnp.float32)
                    + b_ref[...]) * dinv


def _tc4_body(d0_ref, d1_ref, s0_ref, s1_ref, hs_ref,
              w1_ref, b1_ref, w2_ref, b2_ref, out_ref):
    dinv = lax.rsqrt(d0_ref[...] + d1_ref[...] + 1.0)
    embed = (s0_ref[...] + s1_ref[...] + hs_ref[...]) * dinv
    hidden = jnp.maximum(
        jnp.dot(embed, w1_ref[...], preferred_element_type=jnp.float32)
        + b1_ref[...], 0.0)
    out_ref[...] = (jnp.dot(hidden, w2_ref[...],
                            preferred_element_type=jnp.float32) + b2_ref[...])


def _row_spec():
    return pl.BlockSpec((_BLK, _D), lambda i: (i, 0))


def _deg_spec():
    return pl.BlockSpec((_BLK, 1), lambda i: (i, 0))


def _full_spec(shape):
    return pl.BlockSpec(shape, lambda i: tuple(0 for _ in shape))


def kernel(x, edge_index, emb1, emb3, learnable_x, cond_Wi, cond_bi, cond_Wo,
           cond_bo, g_W1, g_b1, g_W2, g_b2, c_W1, c_b1, c_W2, c_b2):
    n, d = x.shape
    grid = (n // _BLK,)

    degp = _deg_kernel(edge_index)
    d0 = degp[0, :, 0].reshape(n, 1)
    d1 = degp[1, :, 0].reshape(n, 1)

    x_out, h1_pre = pl.pallas_call(
        _tc1_body,
        grid=grid,
        in_specs=[_row_spec(), _row_spec(),
                  _full_spec((_D, _D)), _full_spec((1, _D))],
        out_specs=[_row_spec(), _row_spec()],
        out_shape=[jax.ShapeDtypeStruct((n, d), jnp.float32)] * 2,
    )(x, learnable_x, g_W1, g_b1.reshape(1, d))

    hs1 = pl.pallas_call(
        _tc2_body,
        grid=grid,
        in_specs=[_deg_spec(), _deg_spec(), _row_spec()],
        out_specs=_row_spec(),
        out_shape=jax.ShapeDtypeStruct((n, d), jnp.float32),
    )(d0, d1, h1_pre)

    s1 = _segsum_kernel(hs1, edge_index)

    hs2 = pl.pallas_call(
        _tc3_body,
        grid=grid,
        in_specs=[_deg_spec(), _deg_spec(), _row_spec(), _row_spec(),
                  _row_spec(), _full_spec((_D, _D)), _full_spec((1, _D))],
        out_specs=_row_spec(),
        out_shape=jax.ShapeDtypeStruct((n, d), jnp.float32),
    )(d0, d1, s1[0], s1[1], hs1, g_W2, g_b2.reshape(1, d))

    s2 = _segsum_kernel(hs2, edge_index)

    nh = c_W1.shape[1]
    nc = c_W2.shape[1]
    logits = pl.pallas_call(
        _tc4_body,
        grid=grid,
        in_specs=[_deg_spec(), _deg_spec(), _row_spec(), _row_spec(),
                  _row_spec(), _full_spec((_D, nh)), _full_spec((1, nh)),
                  _full_spec((nh, nc)), _full_spec((1, nc))],
        out_specs=pl.BlockSpec((_BLK, nc), lambda i: (i, 0)),
        out_shape=jax.ShapeDtypeStruct((n, nc), jnp.float32),
    )(d0, d1, s2[0], s2[1], hs2, c_W1, c_b1.reshape(1, nh),
      c_W2, c_b2.reshape(1, nc))

    return (x_out, logits)
